# Initial kernel scaffold; baseline (speedup 1.0000x reference)
import jax
import jax.numpy as jnp
from jax.experimental import pallas as pl


def kernel(z, embedding_weight):
    raise NotImplementedError("write your pallas kernel here")


# speedup vs baseline: 1.0718x; 1.0718x over previous
import functools

import jax
import jax.numpy as jnp
from jax import lax
from jax.experimental import pallas as pl
from jax.experimental.pallas import tpu as pltpu
from jax.experimental.pallas import tpu_sc as plsc

_N_EMB = 8192
_DIM = 256
_ROWS = 16384
_TR = 256
_N_TILES = _ROWS // _TR
_BOUNDS = (0, 2736, 5472, 8192)


def _argmin_body(z_ref, s_ref, e_ref, idx_ref):
    zb = z_ref[...]
    s = s_ref[...]
    accv = None
    acci = None
    for g in range(len(_BOUNDS) - 1):
        lo, hi = _BOUNDS[g], _BOUNDS[g + 1]
        ec = e_ref[lo:hi, :]
        mm = lax.dot_general(zb, ec, (((1,), (1,)), ((), ())),
                             preferred_element_type=jnp.float32)
        dch = s - 2.0 * mm
        m = jnp.min(dch, axis=1, keepdims=True)
        cols = lax.broadcasted_iota(jnp.int32, dch.shape, 1) + lo
        c = jnp.min(jnp.where(dch == m, cols, jnp.int32(2 ** 30)),
                    axis=1, keepdims=True)
        mq = m.astype(jnp.bfloat16).astype(jnp.float32)
        if g == 0:
            accv, acci = mq, c
        else:
            better = m < accv
            accv = jnp.where(better, mq, accv)
            acci = jnp.where(better, c, acci)
    idx_ref[...] = acci


_argmin_call = pl.pallas_call(
    _argmin_body,
    grid=(_N_TILES,),
    in_specs=[
        pl.BlockSpec((_TR, _DIM), lambda i: (i, 0)),
        pl.BlockSpec((_TR, 1), lambda i: (i, 0)),
        pl.BlockSpec((_N_EMB, _DIM), lambda i: (0, 0)),
    ],
    out_specs=pl.BlockSpec((_TR, 1), lambda i: (i, 0)),
    out_shape=jax.ShapeDtypeStruct((_ROWS, 1), jnp.int32),
    compiler_params=pltpu.CompilerParams(
        dimension_semantics=("arbitrary",)),
)


def _st_loss_body(z_ref, q_ref, st_ref, loss_ref):
    i = pl.program_id(0)
    zb = z_ref[...]
    qb = q_ref[...]
    diff = qb - zb
    st_ref[...] = zb + diff

    @pl.when(i == 0)
    def _():
        loss_ref[0, 0] = 0.0

    loss_ref[0, 0] += jnp.sum(diff * diff)


_st_loss_call = pl.pallas_call(
    _st_loss_body,
    grid=(_N_TILES,),
    in_specs=[
        pl.BlockSpec((_TR, _DIM), lambda i: (i, 0)),
        pl.BlockSpec((_TR, _DIM), lambda i: (i, 0)),
    ],
    out_specs=[
        pl.BlockSpec((_TR, _DIM), lambda i: (i, 0)),
        pl.BlockSpec(memory_space=pltpu.SMEM, block_shape=(1, 1),
                     index_map=lambda i: (0, 0)),
    ],
    out_shape=[
        jax.ShapeDtypeStruct((_ROWS, _DIM), jnp.float32),
        jax.ShapeDtypeStruct((1, 1), jnp.float32),
    ],
    compiler_params=pltpu.CompilerParams(
        dimension_semantics=("arbitrary",)),
)


@functools.lru_cache(maxsize=1)
def _make_sc_gather():
    info = plsc.get_sparse_core_info()
    nc, ns = info.num_cores, info.num_subcores
    nw = nc * ns
    b_per_w = _ROWS // nw
    chunk = 128
    n_chunks = b_per_w // chunk
    mesh = plsc.VectorSubcoreMesh(core_axis_name="c", subcore_axis_name="s")

    @functools.partial(
        pl.kernel, mesh=mesh,
        out_type=jax.ShapeDtypeStruct((_ROWS, _DIM), jnp.float32),
        scratch_types=[
            pltpu.VMEM((chunk,), jnp.int32),
            pltpu.VMEM((chunk, _DIM), jnp.float32),
            pltpu.SemaphoreType.DMA,
        ],
    )
    def gather_k(idx_hbm, table_hbm, out_hbm, idx_v, rows_v, sem):
        wid = lax.axis_index("s") * nc + lax.axis_index("c")
        base = wid * b_per_w
        for c in range(n_chunks):
            off = base + c * chunk
            pltpu.sync_copy(idx_hbm.at[pl.ds(off, chunk)], idx_v)
            pltpu.async_copy(table_hbm.at[idx_v], rows_v, sem).wait()
            pltpu.sync_copy(rows_v, out_hbm.at[pl.ds(off, chunk)])

    return gather_k


def kernel(z, embedding_weight):
    zf = z.reshape(-1, _DIM)
    s = jnp.sum(zf ** 2, axis=1, keepdims=True)
    idx = _argmin_call(zf, s, embedding_weight)
    q = _make_sc_gather()(idx.reshape(-1), embedding_weight)
    st, loss_sum = _st_loss_call(zf, q)
    m = loss_sum[0, 0] / jnp.float32(_ROWS * _DIM)
    loss = m + 0.25 * m
    return st.reshape(z.shape), loss
